# baseline (speedup 1.0000x reference)
import functools

import jax
import jax.numpy as jnp
from jax import lax
from jax.experimental import pallas as pl
from jax.experimental.pallas import tpu as pltpu
from jax.experimental.pallas import tpu_sc as plsc
from jax._src.pallas import mpmd as _mpmd

L = 16
NC = 2
NS = 16
NW = NC * NS

CH = 1024
NBUF = 8


def _make_tc_bulk_copy(n_rows: int, d: int):
    nchunk = n_rows // CH

    def body(d_hbm, o_hbm, dbuf, dsem, osem):
        def in_d(c, s):
            return pltpu.make_async_copy(
                d_hbm.at[pl.ds(c * CH, CH), :], dbuf.at[s], dsem.at[s])

        def out_o(c, s):
            return pltpu.make_async_copy(
                dbuf.at[s], o_hbm.at[pl.ds(c * CH, CH), :], osem.at[s])

        for s in range(NBUF):
            in_d(s, s).start()

        for c in range(nchunk):
            s = c % NBUF
            in_d(c, s).wait()
            out_o(c, s).start()
            nxt = c + NBUF
            if nxt < nchunk:
                out_o(c, s).wait()
                in_d(nxt, s).start()

        for c in range(max(nchunk - NBUF, 0), nchunk):
            if c + NBUF >= nchunk:
                out_o(c, c % NBUF).wait()

    return pl.pallas_call(
        body,
        in_specs=[pl.BlockSpec(memory_space=pl.ANY)],
        out_specs=pl.BlockSpec(memory_space=pl.ANY),
        out_shape=jax.ShapeDtypeStruct((n_rows, d), jnp.float32),
        scratch_shapes=[
            pltpu.VMEM((NBUF, CH, d), jnp.float32),
            pltpu.SemaphoreType.DMA((NBUF,)),
            pltpu.SemaphoreType.DMA((NBUF,)),
        ],
    )


def _make_sc_shift(n_rows: int, d: int, n_sel: int):
    per_w = n_sel // NW
    mesh = plsc.VectorSubcoreMesh(
        core_axis_name="c", subcore_axis_name="s",
        num_cores=NC, num_subcores=NS)

    def body(bulk_hbm, data_hbm, sel_hbm, bias_hbm, out_hbm,
             idx_v, rows_v, bias_v, gsem, ssem):
        del bulk_hbm
        wid = lax.axis_index("s") * NC + lax.axis_index("c")
        base = wid * per_w
        pltpu.sync_copy(sel_hbm.at[pl.ds(base, per_w)], idx_v)
        pltpu.sync_copy(bias_hbm, bias_v)
        pltpu.async_copy(data_hbm.at[idx_v], rows_v, gsem).wait()
        bval = bias_v[...]

        def add_row(r, carry):
            for c in range(d // L):
                rows_v[r, pl.ds(c * L, L)] = rows_v[r, pl.ds(c * L, L)] + bval
            return carry

        lax.fori_loop(0, per_w, add_row, 0)
        pltpu.async_copy(rows_v, out_hbm.at[idx_v], ssem).wait()

    return _mpmd._mpmd_map(
        [(mesh, body)],
        jax.ShapeDtypeStruct((n_rows, d), jnp.float32),
        input_output_aliases={0: 0},
        scratch_types=[
            pltpu.VMEM((per_w,), jnp.int32),
            pltpu.VMEM((per_w, d), jnp.float32),
            pltpu.VMEM((L,), jnp.float32),
            pltpu.SemaphoreType.DMA,
            pltpu.SemaphoreType.DMA,
        ],
        compiler_params=pltpu.CompilerParams(needs_layout_passes=False),
    )


def kernel(data, selection, bias):
    n_rows, d = data.shape
    n_sel = selection.shape[0]
    bias16 = jnp.full((L,), bias, dtype=jnp.float32)

    bulk = _make_tc_bulk_copy(n_rows, d)(data)
    out = _make_sc_shift(n_rows, d, n_sel)(bulk, data, selection, bias16)
    return out


# speedup vs baseline: 1.0094x; 1.0094x over previous
import functools

import jax
import jax.numpy as jnp
from jax import lax
from jax.experimental import pallas as pl
from jax.experimental.pallas import tpu as pltpu
from jax.experimental.pallas import tpu_sc as plsc
from jax._src.pallas import mpmd as _mpmd

L = 16
NC = 2
NS = 16
NW = NC * NS

CH = 1024
NBUF = 8


def _make_tc_bulk_copy(n_rows: int, d: int):
    nchunk = n_rows // CH

    def body(d_hbm, o_hbm, dbuf, dsem, osem):
        def in_d(c, s):
            return pltpu.make_async_copy(
                d_hbm.at[pl.ds(c * CH, CH), :], dbuf.at[s], dsem.at[s])

        def out_o(c, s):
            return pltpu.make_async_copy(
                dbuf.at[s], o_hbm.at[pl.ds(c * CH, CH), :], osem.at[s])

        for s in range(NBUF):
            in_d(s, s).start()

        for c in range(nchunk):
            s = c % NBUF
            in_d(c, s).wait()
            out_o(c, s).start()
            nxt = c + NBUF
            if nxt < nchunk:
                out_o(c, s).wait()
                in_d(nxt, s).start()

        for c in range(max(nchunk - NBUF, 0), nchunk):
            if c + NBUF >= nchunk:
                out_o(c, c % NBUF).wait()

    return pl.pallas_call(
        body,
        in_specs=[pl.BlockSpec(memory_space=pl.ANY)],
        out_specs=pl.BlockSpec(memory_space=pl.ANY),
        out_shape=jax.ShapeDtypeStruct((n_rows, d), jnp.float32),
        scratch_shapes=[
            pltpu.VMEM((NBUF, CH, d), jnp.float32),
            pltpu.SemaphoreType.DMA((NBUF,)),
            pltpu.SemaphoreType.DMA((NBUF,)),
        ],
    )


def _make_sc_gather_bias(d: int, n_sel: int):
    per_w = n_sel // NW
    mesh = plsc.VectorSubcoreMesh(
        core_axis_name="c", subcore_axis_name="s",
        num_cores=NC, num_subcores=NS)

    @functools.partial(
        pl.kernel,
        mesh=mesh,
        out_type=jax.ShapeDtypeStruct((n_sel, d), jnp.float32),
        scratch_types=[
            pltpu.VMEM((per_w,), jnp.int32),
            pltpu.VMEM((per_w, d), jnp.float32),
            pltpu.VMEM((L,), jnp.float32),
            pltpu.SemaphoreType.DMA,
        ],
        compiler_params=pltpu.CompilerParams(needs_layout_passes=False),
    )
    def gather_bias(data_hbm, sel_hbm, bias_hbm, shifted_hbm,
                    idx_v, rows_v, bias_v, gsem):
        wid = lax.axis_index("s") * NC + lax.axis_index("c")
        base = wid * per_w
        pltpu.sync_copy(sel_hbm.at[pl.ds(base, per_w)], idx_v)
        pltpu.sync_copy(bias_hbm, bias_v)
        pltpu.async_copy(data_hbm.at[idx_v], rows_v, gsem).wait()
        bval = bias_v[...]

        def add_row(r, carry):
            for c in range(d // L):
                rows_v[r, pl.ds(c * L, L)] = rows_v[r, pl.ds(c * L, L)] + bval
            return carry

        lax.fori_loop(0, per_w, add_row, 0)
        pltpu.sync_copy(rows_v, shifted_hbm.at[pl.ds(base, per_w), :])

    return gather_bias


def _make_sc_scatter(n_rows: int, d: int, n_sel: int):
    per_w = n_sel // NW
    mesh = plsc.VectorSubcoreMesh(
        core_axis_name="c", subcore_axis_name="s",
        num_cores=NC, num_subcores=NS)

    def body(bulk_hbm, sel_hbm, shifted_hbm, out_hbm, idx_v, rows_v, ssem):
        del bulk_hbm
        wid = lax.axis_index("s") * NC + lax.axis_index("c")
        base = wid * per_w
        pltpu.sync_copy(sel_hbm.at[pl.ds(base, per_w)], idx_v)
        pltpu.sync_copy(shifted_hbm.at[pl.ds(base, per_w), :], rows_v)
        pltpu.async_copy(rows_v, out_hbm.at[idx_v], ssem).wait()

    return _mpmd._mpmd_map(
        [(mesh, body)],
        jax.ShapeDtypeStruct((n_rows, d), jnp.float32),
        input_output_aliases={0: 0},
        scratch_types=[
            pltpu.VMEM((per_w,), jnp.int32),
            pltpu.VMEM((per_w, d), jnp.float32),
            pltpu.SemaphoreType.DMA,
        ],
        compiler_params=pltpu.CompilerParams(needs_layout_passes=False),
    )


def kernel(data, selection, bias):
    n_rows, d = data.shape
    n_sel = selection.shape[0]
    bias16 = jnp.full((L,), bias, dtype=jnp.float32)

    shifted = _make_sc_gather_bias(d, n_sel)(data, selection, bias16)
    bulk = _make_tc_bulk_copy(n_rows, d)(data)
    out = _make_sc_scatter(n_rows, d, n_sel)(bulk, selection, shifted)
    return out


# speedup vs baseline: 1.2711x; 1.2593x over previous
import functools

import jax
import jax.numpy as jnp
from jax import lax
from jax.experimental import pallas as pl
from jax.experimental.pallas import tpu as pltpu
from jax.experimental.pallas import tpu_sc as plsc
from jax._src.pallas import mpmd as _mpmd

L = 16
NC = 2
NS = 16
NW = NC * NS

CH = 512
NBUF = 16
PF = NBUF // 2


def _make_tc_bulk_copy(n_rows: int, d: int):
    nchunk = n_rows // CH

    def body(d_hbm, o_hbm, dbuf, dsem, osem):
        def in_d(c):
            s = c % NBUF
            return pltpu.make_async_copy(
                d_hbm.at[pl.ds(c * CH, CH), :], dbuf.at[s], dsem.at[s])

        def out_o(c):
            s = c % NBUF
            return pltpu.make_async_copy(
                dbuf.at[s], o_hbm.at[pl.ds(c * CH, CH), :], osem.at[s])

        for c in range(min(PF, nchunk)):
            in_d(c).start()

        for c in range(nchunk):
            in_d(c).wait()
            out_o(c).start()
            k = c + PF
            if k < nchunk:
                fr = k - NBUF
                if fr >= 0:
                    out_o(fr).wait()
                in_d(k).start()

        for c in range(max(nchunk - NBUF, 0), nchunk):
            out_o(c).wait()

    return pl.pallas_call(
        body,
        in_specs=[pl.BlockSpec(memory_space=pl.ANY)],
        out_specs=pl.BlockSpec(memory_space=pl.ANY),
        out_shape=jax.ShapeDtypeStruct((n_rows, d), jnp.float32),
        scratch_shapes=[
            pltpu.VMEM((NBUF, CH, d), jnp.float32),
            pltpu.SemaphoreType.DMA((NBUF,)),
            pltpu.SemaphoreType.DMA((NBUF,)),
        ],
    )


def _make_sc_gather_bias(d: int, n_sel: int):
    per_w = n_sel // NW
    mesh = plsc.VectorSubcoreMesh(
        core_axis_name="c", subcore_axis_name="s",
        num_cores=NC, num_subcores=NS)

    @functools.partial(
        pl.kernel,
        mesh=mesh,
        out_type=jax.ShapeDtypeStruct((n_sel, d), jnp.float32),
        scratch_types=[
            pltpu.VMEM((per_w,), jnp.int32),
            pltpu.VMEM((per_w, d), jnp.float32),
            pltpu.VMEM((L,), jnp.float32),
            pltpu.SemaphoreType.DMA,
        ],
        compiler_params=pltpu.CompilerParams(needs_layout_passes=False),
    )
    def gather_bias(data_hbm, sel_hbm, bias_hbm, shifted_hbm,
                    idx_v, rows_v, bias_v, gsem):
        wid = lax.axis_index("s") * NC + lax.axis_index("c")
        base = wid * per_w
        pltpu.sync_copy(sel_hbm.at[pl.ds(base, per_w)], idx_v)
        pltpu.sync_copy(bias_hbm, bias_v)
        pltpu.async_copy(data_hbm.at[idx_v], rows_v, gsem).wait()
        bval = bias_v[...]

        def add_row(r, carry):
            for c in range(d // L):
                rows_v[r, pl.ds(c * L, L)] = rows_v[r, pl.ds(c * L, L)] + bval
            return carry

        lax.fori_loop(0, per_w, add_row, 0)
        pltpu.sync_copy(rows_v, shifted_hbm.at[pl.ds(base, per_w), :])

    return gather_bias


def _make_sc_scatter(n_rows: int, d: int, n_sel: int):
    per_w = n_sel // NW
    mesh = plsc.VectorSubcoreMesh(
        core_axis_name="c", subcore_axis_name="s",
        num_cores=NC, num_subcores=NS)

    def body(bulk_hbm, sel_hbm, shifted_hbm, out_hbm, idx_v, rows_v, ssem):
        del bulk_hbm
        wid = lax.axis_index("s") * NC + lax.axis_index("c")
        base = wid * per_w
        pltpu.sync_copy(sel_hbm.at[pl.ds(base, per_w)], idx_v)
        pltpu.sync_copy(shifted_hbm.at[pl.ds(base, per_w), :], rows_v)
        pltpu.async_copy(rows_v, out_hbm.at[idx_v], ssem).wait()

    return _mpmd._mpmd_map(
        [(mesh, body)],
        jax.ShapeDtypeStruct((n_rows, d), jnp.float32),
        input_output_aliases={0: 0},
        scratch_types=[
            pltpu.VMEM((per_w,), jnp.int32),
            pltpu.VMEM((per_w, d), jnp.float32),
            pltpu.SemaphoreType.DMA,
        ],
        compiler_params=pltpu.CompilerParams(needs_layout_passes=False),
    )


def kernel(data, selection, bias):
    n_rows, d = data.shape
    n_sel = selection.shape[0]
    bias16 = jnp.full((L,), bias, dtype=jnp.float32)

    shifted = _make_sc_gather_bias(d, n_sel)(data, selection, bias16)
    bulk = _make_tc_bulk_copy(n_rows, d)(data)
    out = _make_sc_scatter(n_rows, d, n_sel)(bulk, selection, shifted)
    return out


# speedup vs baseline: 1.2723x; 1.0009x over previous
import functools

import jax
import jax.numpy as jnp
from jax import lax
from jax.experimental import pallas as pl
from jax.experimental.pallas import tpu as pltpu
from jax.experimental.pallas import tpu_sc as plsc
from jax._src.pallas import mpmd as _mpmd

L = 16
NC = 2
NS = 16
NW = NC * NS

CH = 256
NBUF = 32
PF = NBUF // 2


def _make_tc_bulk_copy(n_rows: int, d: int):
    nchunk = n_rows // CH

    def body(d_hbm, o_hbm, dbuf, dsem, osem):
        def in_d(c):
            s = c % NBUF
            return pltpu.make_async_copy(
                d_hbm.at[pl.ds(c * CH, CH), :], dbuf.at[s], dsem.at[s])

        def out_o(c):
            s = c % NBUF
            return pltpu.make_async_copy(
                dbuf.at[s], o_hbm.at[pl.ds(c * CH, CH), :], osem.at[s])

        for c in range(min(PF, nchunk)):
            in_d(c).start()

        for c in range(nchunk):
            in_d(c).wait()
            out_o(c).start()
            k = c + PF
            if k < nchunk:
                fr = k - NBUF
                if fr >= 0:
                    out_o(fr).wait()
                in_d(k).start()

        for c in range(max(nchunk - NBUF, 0), nchunk):
            out_o(c).wait()

    return pl.pallas_call(
        body,
        in_specs=[pl.BlockSpec(memory_space=pl.ANY)],
        out_specs=pl.BlockSpec(memory_space=pl.ANY),
        out_shape=jax.ShapeDtypeStruct((n_rows, d), jnp.float32),
        scratch_shapes=[
            pltpu.VMEM((NBUF, CH, d), jnp.float32),
            pltpu.SemaphoreType.DMA((NBUF,)),
            pltpu.SemaphoreType.DMA((NBUF,)),
        ],
    )


def _make_sc_gather_bias(d: int, n_sel: int):
    per_w = n_sel // NW
    mesh = plsc.VectorSubcoreMesh(
        core_axis_name="c", subcore_axis_name="s",
        num_cores=NC, num_subcores=NS)

    @functools.partial(
        pl.kernel,
        mesh=mesh,
        out_type=jax.ShapeDtypeStruct((n_sel, d), jnp.float32),
        scratch_types=[
            pltpu.VMEM((per_w,), jnp.int32),
            pltpu.VMEM((per_w, d), jnp.float32),
            pltpu.VMEM((L,), jnp.float32),
            pltpu.SemaphoreType.DMA,
        ],
        compiler_params=pltpu.CompilerParams(needs_layout_passes=False),
    )
    def gather_bias(data_hbm, sel_hbm, bias_hbm, shifted_hbm,
                    idx_v, rows_v, bias_v, gsem):
        wid = lax.axis_index("s") * NC + lax.axis_index("c")
        base = wid * per_w
        pltpu.sync_copy(sel_hbm.at[pl.ds(base, per_w)], idx_v)
        pltpu.sync_copy(bias_hbm, bias_v)
        pltpu.async_copy(data_hbm.at[idx_v], rows_v, gsem).wait()
        bval = bias_v[...]

        def add_row(r, carry):
            for c in range(d // L):
                rows_v[r, pl.ds(c * L, L)] = rows_v[r, pl.ds(c * L, L)] + bval
            return carry

        lax.fori_loop(0, per_w, add_row, 0)
        pltpu.sync_copy(rows_v, shifted_hbm.at[pl.ds(base, per_w), :])

    return gather_bias


def _make_sc_scatter(n_rows: int, d: int, n_sel: int):
    per_w = n_sel // NW
    mesh = plsc.VectorSubcoreMesh(
        core_axis_name="c", subcore_axis_name="s",
        num_cores=NC, num_subcores=NS)

    def body(bulk_hbm, sel_hbm, shifted_hbm, out_hbm, idx_v, rows_v, ssem):
        del bulk_hbm
        wid = lax.axis_index("s") * NC + lax.axis_index("c")
        base = wid * per_w
        pltpu.sync_copy(sel_hbm.at[pl.ds(base, per_w)], idx_v)
        pltpu.sync_copy(shifted_hbm.at[pl.ds(base, per_w), :], rows_v)
        pltpu.async_copy(rows_v, out_hbm.at[idx_v], ssem).wait()

    return _mpmd._mpmd_map(
        [(mesh, body)],
        jax.ShapeDtypeStruct((n_rows, d), jnp.float32),
        input_output_aliases={0: 0},
        scratch_types=[
            pltpu.VMEM((per_w,), jnp.int32),
            pltpu.VMEM((per_w, d), jnp.float32),
            pltpu.SemaphoreType.DMA,
        ],
        compiler_params=pltpu.CompilerParams(needs_layout_passes=False),
    )


def kernel(data, selection, bias):
    n_rows, d = data.shape
    n_sel = selection.shape[0]
    bias16 = jnp.full((L,), bias, dtype=jnp.float32)

    shifted = _make_sc_gather_bias(d, n_sel)(data, selection, bias16)
    bulk = _make_tc_bulk_copy(n_rows, d)(data)
    out = _make_sc_scatter(n_rows, d, n_sel)(bulk, selection, shifted)
    return out


# speedup vs baseline: 1.2966x; 1.0191x over previous
import functools

import jax
import jax.numpy as jnp
from jax import lax
from jax.experimental import pallas as pl
from jax.experimental.pallas import tpu as pltpu
from jax.experimental.pallas import tpu_sc as plsc
from jax._src.pallas import mpmd as _mpmd

L = 16
NC = 2
NS = 16
NW = NC * NS

CH = 512
NBUF = 16
PF = 12


def _make_tc_bulk_copy(n_rows: int, d: int):
    nchunk = n_rows // CH

    def body(d_hbm, o_hbm, dbuf, dsem, osem):
        def in_d(c):
            s = c % NBUF
            return pltpu.make_async_copy(
                d_hbm.at[pl.ds(c * CH, CH), :], dbuf.at[s], dsem.at[s])

        def out_o(c):
            s = c % NBUF
            return pltpu.make_async_copy(
                dbuf.at[s], o_hbm.at[pl.ds(c * CH, CH), :], osem.at[s])

        for c in range(min(PF, nchunk)):
            in_d(c).start()

        for c in range(nchunk):
            in_d(c).wait()
            out_o(c).start()
            k = c + PF
            if k < nchunk:
                fr = k - NBUF
                if fr >= 0:
                    out_o(fr).wait()
                in_d(k).start()

        for c in range(max(nchunk - NBUF, 0), nchunk):
            out_o(c).wait()

    return pl.pallas_call(
        body,
        in_specs=[pl.BlockSpec(memory_space=pl.ANY)],
        out_specs=pl.BlockSpec(memory_space=pl.ANY),
        out_shape=jax.ShapeDtypeStruct((n_rows, d), jnp.float32),
        scratch_shapes=[
            pltpu.VMEM((NBUF, CH, d), jnp.float32),
            pltpu.SemaphoreType.DMA((NBUF,)),
            pltpu.SemaphoreType.DMA((NBUF,)),
        ],
    )


def _make_sc_gather_bias(d: int, n_sel: int):
    per_w = n_sel // NW
    mesh = plsc.VectorSubcoreMesh(
        core_axis_name="c", subcore_axis_name="s",
        num_cores=NC, num_subcores=NS)

    @functools.partial(
        pl.kernel,
        mesh=mesh,
        out_type=jax.ShapeDtypeStruct((n_sel, d), jnp.float32),
        scratch_types=[
            pltpu.VMEM((per_w,), jnp.int32),
            pltpu.VMEM((per_w, d), jnp.float32),
            pltpu.VMEM((L,), jnp.float32),
            pltpu.SemaphoreType.DMA,
        ],
        compiler_params=pltpu.CompilerParams(needs_layout_passes=False),
    )
    def gather_bias(data_hbm, sel_hbm, bias_hbm, shifted_hbm,
                    idx_v, rows_v, bias_v, gsem):
        wid = lax.axis_index("s") * NC + lax.axis_index("c")
        base = wid * per_w
        pltpu.sync_copy(sel_hbm.at[pl.ds(base, per_w)], idx_v)
        pltpu.sync_copy(bias_hbm, bias_v)
        pltpu.async_copy(data_hbm.at[idx_v], rows_v, gsem).wait()
        bval = bias_v[...]

        def add_row(r, carry):
            for c in range(d // L):
                rows_v[r, pl.ds(c * L, L)] = rows_v[r, pl.ds(c * L, L)] + bval
            return carry

        lax.fori_loop(0, per_w, add_row, 0)
        pltpu.sync_copy(rows_v, shifted_hbm.at[pl.ds(base, per_w), :])

    return gather_bias


def _make_sc_scatter(n_rows: int, d: int, n_sel: int):
    per_w = n_sel // NW
    mesh = plsc.VectorSubcoreMesh(
        core_axis_name="c", subcore_axis_name="s",
        num_cores=NC, num_subcores=NS)

    def body(bulk_hbm, sel_hbm, shifted_hbm, out_hbm, idx_v, rows_v, ssem):
        del bulk_hbm
        wid = lax.axis_index("s") * NC + lax.axis_index("c")
        base = wid * per_w
        pltpu.sync_copy(sel_hbm.at[pl.ds(base, per_w)], idx_v)
        pltpu.sync_copy(shifted_hbm.at[pl.ds(base, per_w), :], rows_v)
        pltpu.async_copy(rows_v, out_hbm.at[idx_v], ssem).wait()

    return _mpmd._mpmd_map(
        [(mesh, body)],
        jax.ShapeDtypeStruct((n_rows, d), jnp.float32),
        input_output_aliases={0: 0},
        scratch_types=[
            pltpu.VMEM((per_w,), jnp.int32),
            pltpu.VMEM((per_w, d), jnp.float32),
            pltpu.SemaphoreType.DMA,
        ],
        compiler_params=pltpu.CompilerParams(needs_layout_passes=False),
    )


def kernel(data, selection, bias):
    n_rows, d = data.shape
    n_sel = selection.shape[0]
    bias16 = jnp.full((L,), bias, dtype=jnp.float32)

    shifted = _make_sc_gather_bias(d, n_sel)(data, selection, bias16)
    bulk = _make_tc_bulk_copy(n_rows, d)(data)
    out = _make_sc_scatter(n_rows, d, n_sel)(bulk, selection, shifted)
    return out


# speedup vs baseline: 1.3022x; 1.0043x over previous
import functools

import jax
import jax.numpy as jnp
from jax import lax
from jax.experimental import pallas as pl
from jax.experimental.pallas import tpu as pltpu
from jax.experimental.pallas import tpu_sc as plsc
from jax._src.pallas import mpmd as _mpmd

L = 16
NC = 2
NS = 16
NW = NC * NS

CH = 512
NBUF = 24
PF = 18


def _make_tc_bulk_copy(n_rows: int, d: int):
    nchunk = n_rows // CH

    def body(d_hbm, o_hbm, dbuf, dsem, osem):
        def in_d(c):
            s = c % NBUF
            return pltpu.make_async_copy(
                d_hbm.at[pl.ds(c * CH, CH), :], dbuf.at[s], dsem.at[s])

        def out_o(c):
            s = c % NBUF
            return pltpu.make_async_copy(
                dbuf.at[s], o_hbm.at[pl.ds(c * CH, CH), :], osem.at[s])

        for c in range(min(PF, nchunk)):
            in_d(c).start()

        for c in range(nchunk):
            in_d(c).wait()
            out_o(c).start()
            k = c + PF
            if k < nchunk:
                fr = k - NBUF
                if fr >= 0:
                    out_o(fr).wait()
                in_d(k).start()

        for c in range(max(nchunk - NBUF, 0), nchunk):
            out_o(c).wait()

    return pl.pallas_call(
        body,
        in_specs=[pl.BlockSpec(memory_space=pl.ANY)],
        out_specs=pl.BlockSpec(memory_space=pl.ANY),
        out_shape=jax.ShapeDtypeStruct((n_rows, d), jnp.float32),
        scratch_shapes=[
            pltpu.VMEM((NBUF, CH, d), jnp.float32),
            pltpu.SemaphoreType.DMA((NBUF,)),
            pltpu.SemaphoreType.DMA((NBUF,)),
        ],
    )


def _make_sc_gather_bias(d: int, n_sel: int):
    per_w = n_sel // NW
    mesh = plsc.VectorSubcoreMesh(
        core_axis_name="c", subcore_axis_name="s",
        num_cores=NC, num_subcores=NS)

    @functools.partial(
        pl.kernel,
        mesh=mesh,
        out_type=jax.ShapeDtypeStruct((n_sel, d), jnp.float32),
        scratch_types=[
            pltpu.VMEM((per_w,), jnp.int32),
            pltpu.VMEM((per_w, d), jnp.float32),
            pltpu.VMEM((L,), jnp.float32),
            pltpu.SemaphoreType.DMA,
        ],
        compiler_params=pltpu.CompilerParams(needs_layout_passes=False),
    )
    def gather_bias(data_hbm, sel_hbm, bias_hbm, shifted_hbm,
                    idx_v, rows_v, bias_v, gsem):
        wid = lax.axis_index("s") * NC + lax.axis_index("c")
        base = wid * per_w
        pltpu.sync_copy(sel_hbm.at[pl.ds(base, per_w)], idx_v)
        pltpu.sync_copy(bias_hbm, bias_v)
        pltpu.async_copy(data_hbm.at[idx_v], rows_v, gsem).wait()
        bval = bias_v[...]

        def add_row(r, carry):
            for c in range(d // L):
                rows_v[r, pl.ds(c * L, L)] = rows_v[r, pl.ds(c * L, L)] + bval
            return carry

        lax.fori_loop(0, per_w, add_row, 0)
        pltpu.sync_copy(rows_v, shifted_hbm.at[pl.ds(base, per_w), :])

    return gather_bias


def _make_sc_scatter(n_rows: int, d: int, n_sel: int):
    per_w = n_sel // NW
    mesh = plsc.VectorSubcoreMesh(
        core_axis_name="c", subcore_axis_name="s",
        num_cores=NC, num_subcores=NS)

    def body(bulk_hbm, sel_hbm, shifted_hbm, out_hbm, idx_v, rows_v, ssem):
        del bulk_hbm
        wid = lax.axis_index("s") * NC + lax.axis_index("c")
        base = wid * per_w
        pltpu.sync_copy(sel_hbm.at[pl.ds(base, per_w)], idx_v)
        pltpu.sync_copy(shifted_hbm.at[pl.ds(base, per_w), :], rows_v)
        pltpu.async_copy(rows_v, out_hbm.at[idx_v], ssem).wait()

    return _mpmd._mpmd_map(
        [(mesh, body)],
        jax.ShapeDtypeStruct((n_rows, d), jnp.float32),
        input_output_aliases={0: 0},
        scratch_types=[
            pltpu.VMEM((per_w,), jnp.int32),
            pltpu.VMEM((per_w, d), jnp.float32),
            pltpu.SemaphoreType.DMA,
        ],
        compiler_params=pltpu.CompilerParams(needs_layout_passes=False),
    )


def kernel(data, selection, bias):
    n_rows, d = data.shape
    n_sel = selection.shape[0]
    bias16 = jnp.full((L,), bias, dtype=jnp.float32)

    shifted = _make_sc_gather_bias(d, n_sel)(data, selection, bias16)
    bulk = _make_tc_bulk_copy(n_rows, d)(data)
    out = _make_sc_scatter(n_rows, d, n_sel)(bulk, selection, shifted)
    return out


# speedup vs baseline: 1.3050x; 1.0022x over previous
import functools

import jax
import jax.numpy as jnp
from jax import lax
from jax.experimental import pallas as pl
from jax.experimental.pallas import tpu as pltpu
from jax.experimental.pallas import tpu_sc as plsc
from jax._src.pallas import mpmd as _mpmd

L = 16
NC = 2
NS = 16
NW = NC * NS

CH = 1024
NBUF = 12
PF = 9


def _make_tc_bulk_copy(n_rows: int, d: int):
    nchunk = n_rows // CH

    def body(d_hbm, o_hbm, dbuf, dsem, osem):
        def in_d(c):
            s = c % NBUF
            return pltpu.make_async_copy(
                d_hbm.at[pl.ds(c * CH, CH), :], dbuf.at[s], dsem.at[s])

        def out_o(c):
            s = c % NBUF
            return pltpu.make_async_copy(
                dbuf.at[s], o_hbm.at[pl.ds(c * CH, CH), :], osem.at[s])

        for c in range(min(PF, nchunk)):
            in_d(c).start()

        for c in range(nchunk):
            in_d(c).wait()
            out_o(c).start()
            k = c + PF
            if k < nchunk:
                fr = k - NBUF
                if fr >= 0:
                    out_o(fr).wait()
                in_d(k).start()

        for c in range(max(nchunk - NBUF, 0), nchunk):
            out_o(c).wait()

    return pl.pallas_call(
        body,
        in_specs=[pl.BlockSpec(memory_space=pl.ANY)],
        out_specs=pl.BlockSpec(memory_space=pl.ANY),
        out_shape=jax.ShapeDtypeStruct((n_rows, d), jnp.float32),
        scratch_shapes=[
            pltpu.VMEM((NBUF, CH, d), jnp.float32),
            pltpu.SemaphoreType.DMA((NBUF,)),
            pltpu.SemaphoreType.DMA((NBUF,)),
        ],
    )


def _make_sc_gather_bias(d: int, n_sel: int):
    per_w = n_sel // NW
    mesh = plsc.VectorSubcoreMesh(
        core_axis_name="c", subcore_axis_name="s",
        num_cores=NC, num_subcores=NS)

    @functools.partial(
        pl.kernel,
        mesh=mesh,
        out_type=jax.ShapeDtypeStruct((n_sel, d), jnp.float32),
        scratch_types=[
            pltpu.VMEM((per_w,), jnp.int32),
            pltpu.VMEM((per_w, d), jnp.float32),
            pltpu.VMEM((L,), jnp.float32),
            pltpu.SemaphoreType.DMA,
        ],
        compiler_params=pltpu.CompilerParams(needs_layout_passes=False),
    )
    def gather_bias(data_hbm, sel_hbm, bias_hbm, shifted_hbm,
                    idx_v, rows_v, bias_v, gsem):
        wid = lax.axis_index("s") * NC + lax.axis_index("c")
        base = wid * per_w
        pltpu.sync_copy(sel_hbm.at[pl.ds(base, per_w)], idx_v)
        pltpu.sync_copy(bias_hbm, bias_v)
        pltpu.async_copy(data_hbm.at[idx_v], rows_v, gsem).wait()
        bval = bias_v[...]

        def add_row(r, carry):
            for c in range(d // L):
                rows_v[r, pl.ds(c * L, L)] = rows_v[r, pl.ds(c * L, L)] + bval
            return carry

        lax.fori_loop(0, per_w, add_row, 0, unroll=8)
        pltpu.sync_copy(rows_v, shifted_hbm.at[pl.ds(base, per_w), :])

    return gather_bias


def _make_sc_scatter(n_rows: int, d: int, n_sel: int):
    per_w = n_sel // NW
    mesh = plsc.VectorSubcoreMesh(
        core_axis_name="c", subcore_axis_name="s",
        num_cores=NC, num_subcores=NS)

    def body(bulk_hbm, sel_hbm, shifted_hbm, out_hbm, idx_v, rows_v, ssem):
        del bulk_hbm
        wid = lax.axis_index("s") * NC + lax.axis_index("c")
        base = wid * per_w
        pltpu.sync_copy(sel_hbm.at[pl.ds(base, per_w)], idx_v)
        pltpu.sync_copy(shifted_hbm.at[pl.ds(base, per_w), :], rows_v)
        pltpu.async_copy(rows_v, out_hbm.at[idx_v], ssem).wait()

    return _mpmd._mpmd_map(
        [(mesh, body)],
        jax.ShapeDtypeStruct((n_rows, d), jnp.float32),
        input_output_aliases={0: 0},
        scratch_types=[
            pltpu.VMEM((per_w,), jnp.int32),
            pltpu.VMEM((per_w, d), jnp.float32),
            pltpu.SemaphoreType.DMA,
        ],
        compiler_params=pltpu.CompilerParams(needs_layout_passes=False),
    )


def kernel(data, selection, bias):
    n_rows, d = data.shape
    n_sel = selection.shape[0]
    bias16 = jnp.full((L,), bias, dtype=jnp.float32)

    shifted = _make_sc_gather_bias(d, n_sel)(data, selection, bias16)
    bulk = _make_tc_bulk_copy(n_rows, d)(data)
    out = _make_sc_scatter(n_rows, d, n_sel)(bulk, selection, shifted)
    return out
